# Initial kernel scaffold; baseline (speedup 1.0000x reference)
import jax
import jax.numpy as jnp
from jax.experimental import pallas as pl


def kernel(x, l1_w1, l1_b1, l1_w2, l1_b2, l1_ws, l1_bs, l2_w1, l2_b1, l2_w2, l2_b2, l2_ws, l2_bs):
    raise NotImplementedError("write your pallas kernel here")


# speedup vs baseline: 65.7368x; 65.7368x over previous
import functools

import jax
import jax.numpy as jnp
import numpy as np
from jax import lax
from jax.experimental import pallas as pl
from jax.experimental.pallas import tpu as pltpu
from jax.experimental.pallas import tpu_sc as plsc

F32 = jnp.float32
K_NB = 20
NEG_INF = float("-inf")

_ATAN_C = (1.0, -0.33333293, 0.19998533, -0.14264892, 0.10958364,
           -0.08427635, 0.05845792, -0.031750634, 0.011257684, -0.0018775737)
_COS_C = (1.0, 2.478392e-09, -0.50000006, 2.2029683e-07, 0.041665923,
          1.5275964e-06, -0.0013909315, 1.843555e-06, 2.3660095e-05,
          4.835589e-07, -4.1198868e-07, 2.3843619e-08, 2.9947572e-15)
_HALF_PI = float(np.pi / 2)
_RAD2DEG = float(180.0 / np.pi)
_TWO_PI_3 = float(2.0 * np.pi / 3.0)


def _poly(c, u):
    acc = jnp.full_like(u, c[-1])
    for k in range(len(c) - 2, -1, -1):
        acc = acc * u + c[k]
    return acc


def _atan_abs(t):
    small = t <= 1.0
    z = jnp.where(small, t, 1.0 / jnp.maximum(t, 1e-30))
    p = z * _poly(_ATAN_C, z * z)
    return jnp.where(small, p, _HALF_PI - p)


def _atan(t):
    return jnp.sign(t) * _atan_abs(jnp.abs(t))


def _acos(z):
    u = jnp.sqrt(jnp.maximum(1.0 - z, 0.0) / jnp.maximum(1.0 + z, 1e-30))
    return 2.0 * _atan_abs(u)


def _cos(v):
    return _poly(_COS_C, v)


def _fmod_floor(a, b):
    return a - b * jnp.floor(a / b)


def _k1_body(xrows_ref, xfull_ref, p_ref, idx_ref, mom_ref):
    b = pl.program_id(0)
    xr = xrows_ref[0]
    xf = xfull_ref[0]
    R = xr.shape[1]
    N = xf.shape[1]
    dot = lax.dot_general(xr.astype(jnp.bfloat16), xf.astype(jnp.bfloat16),
                          (((0,), (0,)), ((), ())),
                          preferred_element_type=F32)
    inner = F32(-2.0) * dot
    xxf = jnp.sum(xf * xf, axis=0, keepdims=True)
    xxr = jnp.sum(xr * xr, axis=0)[:, None]
    nd = (-xxr - inner) - xxf

    lane = lax.broadcasted_iota(jnp.int32, (R, N), 1)
    col = lax.broadcasted_iota(jnp.int32, (R, 32), 1)
    amat = jnp.zeros((R, N), F32)
    idxacc = jnp.zeros((R, 32), jnp.int32)
    for i in range(K_NB):
        v = jnp.max(nd, axis=1, keepdims=True)
        is_m = nd == v
        a = jnp.min(jnp.where(is_m, lane, N), axis=1, keepdims=True)
        oh = lane == a
        amat = jnp.where(oh, F32(1.0), amat)
        idxacc = jnp.where(col == i, a, idxacc)
        nd = jnp.where(oh, F32(NEG_INF), nd)
    idx_ref[0] = idxacc + b * N
    mom_ref[0] = lax.dot_general(amat, p_ref[0], (((1,), (0,)), ((), ())),
                                 preferred_element_type=F32,
                                 precision=lax.Precision.HIGHEST)


def _k1(x, ptab, R=256):
    B, _, N = x.shape
    grid = (B, N // R)
    return pl.pallas_call(
        _k1_body,
        grid=grid,
        in_specs=[
            pl.BlockSpec((1, 3, R), lambda b, r: (b, 0, r)),
            pl.BlockSpec((1, 3, N), lambda b, r: (b, 0, 0)),
            pl.BlockSpec((1, N, 16), lambda b, r: (b, 0, 0)),
        ],
        out_specs=[
            pl.BlockSpec((1, R, 32), lambda b, r: (b, r, 0)),
            pl.BlockSpec((1, R, 16), lambda b, r: (b, r, 0)),
        ],
        out_shape=[
            jax.ShapeDtypeStruct((B, N, 32), jnp.int32),
            jax.ShapeDtypeStruct((B, N, 16), F32),
        ],
    )(x, x, ptab)


def _k2_body(mom_ref, out_ref):
    m = [mom_ref[0, i] for i in range(9)]
    sx, sy, sz, qxx, qyy, qzz, qxy, qxz, qyz = m
    inv_k = F32(1.0 / K_NB)
    mx, my, mz = sx * inv_k, sy * inv_k, sz * inv_k
    g00 = qxx - sx * mx
    g11 = qyy - sy * my
    g22 = qzz - sz * mz
    g01 = qxy - sx * my
    g02 = qxz - sx * mz
    g12 = qyz - sy * mz

    q = (g00 + g11 + g22) * F32(1.0 / 3.0)
    p1 = g01 * g01 + g02 * g02 + g12 * g12
    d0, d1, d2 = g00 - q, g11 - q, g22 - q
    p2 = d0 * d0 + d1 * d1 + d2 * d2 + 2.0 * p1
    p = jnp.sqrt(jnp.maximum(p2 * F32(1.0 / 6.0), 1e-30))
    pinv = 1.0 / p
    b00, b11, b22 = d0 * pinv, d1 * pinv, d2 * pinv
    b01, b02, b12 = g01 * pinv, g02 * pinv, g12 * pinv
    detb = (b00 * (b11 * b22 - b12 * b12)
            - b01 * (b01 * b22 - b12 * b02)
            + b02 * (b01 * b12 - b11 * b02))
    r = jnp.clip(detb * F32(0.5), -1.0, 1.0)
    phi = _acos(r) * F32(1.0 / 3.0)
    l1 = q + 2.0 * p * _cos(phi)
    l3 = q + 2.0 * p * _cos(phi + _TWO_PI_3)
    l2 = 3.0 * q - l1 - l3
    l1 = jnp.maximum(l1, 0.0)
    l2 = jnp.maximum(l2, 0.0)
    l3 = jnp.maximum(l3, 0.0)
    s1, s2, s3 = jnp.sqrt(l1), jnp.sqrt(l2), jnp.sqrt(l3)

    f12 = 1.0 / jnp.maximum(s1 + s2, 1e-20)
    f123 = -1.0 / jnp.maximum((s1 + s2) * (s2 + s3) * (s1 + s3), 1e-30)
    a00, a11, a22 = g00 - l1, g11 - l1, g22 - l1
    c00b, c11b, c22b = g00 - l2, g11 - l2, g22 - l2
    c00 = a00 * c00b + g01 * g01 + g02 * g02
    c11 = g01 * g01 + a11 * c11b + g12 * g12
    c22 = g02 * g02 + g12 * g12 + a22 * c22b
    c01 = a00 * g01 + g01 * c11b + g02 * g12
    c02 = a00 * g02 + g01 * g12 + g02 * c22b
    c12 = g01 * g02 + a11 * g12 + g12 * c22b
    h = [[s1 + f12 * a00 + f123 * c00, f12 * g01 + f123 * c01, f12 * g02 + f123 * c02],
         [None, s1 + f12 * a11 + f123 * c11, f12 * g12 + f123 * c12],
         [None, None, s1 + f12 * a22 + f123 * c22]]
    h[1][0] = h[0][1]
    h[2][0] = h[0][2]
    h[2][1] = h[1][2]

    A = [[h[i][j] for j in range(3)] for i in range(3)]
    one = jnp.ones_like(g00)
    zero = jnp.zeros_like(g00)
    V = [[one if i == j else zero for j in range(3)] for i in range(3)]
    for _ in range(8):
        for (pp, qq) in ((0, 2), (1, 2), (0, 1)):
            apq = A[pp][qq]
            tau = (A[qq][qq] - A[pp][pp]) / (2.0 * apq)
            t = jnp.sign(tau) / (jnp.abs(tau) + jnp.sqrt(1.0 + tau * tau))
            t = jnp.where(apq == 0.0, 0.0, t)
            c = 1.0 / jnp.sqrt(1.0 + t * t)
            s = t * c
            for i in range(3):
                ap, aq = A[i][pp], A[i][qq]
                A[i][pp] = c * ap - s * aq
                A[i][qq] = s * ap + c * aq
            for j in range(3):
                ap, aq = A[pp][j], A[qq][j]
                A[pp][j] = c * ap - s * aq
                A[qq][j] = s * ap + c * aq
            for i in range(3):
                vp, vq = V[i][pp], V[i][qq]
                V[i][pp] = c * vp - s * vq
                V[i][qq] = s * vp + c * vq

    dgs = [A[0][0], A[1][1], A[2][2]]
    m01 = jnp.maximum(dgs[0], dgs[1])
    top1 = dgs[1] > dgs[0]
    top2 = dgs[2] > m01
    def pick(col0, col1, col2):
        return jnp.where(top2, col2, jnp.where(top1, col1, col0))
    gx = pick(V[0][0], V[0][1], V[0][2])
    gy = pick(V[1][0], V[1][1], V[1][2])
    gz = pick(V[2][0], V[2][1], V[2][2])
    dtop = pick(dgs[0], dgs[1], dgs[2])
    mag = jnp.sqrt(jnp.maximum(dtop, 0.0))

    zen = _acos(jnp.clip(gz, -1.0, 1.0)) * F32(_RAD2DEG)
    azi = _atan(gy / gx) * F32(_RAD2DEG)
    width = F32(20.0)
    nbf = F32(9.0)
    outs = {}
    for col, cells0 in ((0, zen), (1, azi)):
        cells = jnp.trunc(cells0)
        cells = jnp.where(cells < 0, cells + F32(180.0), cells)
        bins = _fmod_floor(jnp.floor(cells / width - F32(0.5)), nbf)
        fcent = width * (_fmod_floor(bins + 1.0, nbf) + F32(0.5))
        fv = mag * (_fmod_floor(fcent - cells, F32(180.0)) / width)
        scent = width * (bins + F32(0.5))
        sv = mag * (_fmod_floor(cells - scent, F32(180.0)) / width)
        binsp1 = _fmod_floor(bins + 1.0, nbf)
        for cbin in range(9):
            cb = F32(cbin)
            plane = (jnp.where(bins == cb, fv, 0.0)
                     + jnp.where(binsp1 == cb, sv, 0.0))
            outs[2 * cbin + col] = plane
    zplane = jnp.zeros_like(g00)
    for i in range(32):
        out_ref[0, i] = outs.get(i, zplane)


def _k2(momt):
    B = momt.shape[0]
    return pl.pallas_call(
        _k2_body,
        grid=(B,),
        in_specs=[pl.BlockSpec((1, 16, 16, 128), lambda b: (b, 0, 0, 0))],
        out_specs=pl.BlockSpec((1, 32, 16, 128), lambda b: (b, 0, 0, 0)),
        out_shape=jax.ShapeDtypeStruct((B, 32, 16, 128), F32),
    )(momt)


def _hist_sc(idxflat, table):
    BN = table.shape[0]
    info = plsc.get_sparse_core_info()
    NC, NS = info.num_cores, info.num_subcores
    NW = NC * NS
    PTS = BN // NW
    CH = 16
    NCHUNK = PTS // CH
    mesh = plsc.VectorSubcoreMesh(core_axis_name="c", subcore_axis_name="s")

    @functools.partial(
        pl.kernel, mesh=mesh,
        out_type=jax.ShapeDtypeStruct((BN, 32), F32),
        scratch_types=[
            pltpu.VMEM((CH * K_NB,), jnp.int32),
            pltpu.VMEM((CH * K_NB, 128), F32),
            pltpu.VMEM((CH, 32), F32),
            pltpu.SemaphoreType.DMA,
        ],
    )
    def k(idx_hbm, tab_hbm, out_hbm, idx_v, rows_v, out_v, sem):
        wid = lax.axis_index("s") * NC + lax.axis_index("c")
        base0 = wid * PTS

        def chunk(i, carry):
            base = base0 + i * CH
            pltpu.sync_copy(idx_hbm.at[pl.ds(base * K_NB, CH * K_NB)], idx_v)
            pltpu.async_copy(tab_hbm.at[idx_v], rows_v, sem).wait()
            for ptk in range(CH):
                r0 = ptk * K_NB
                a0 = rows_v[r0, 0:16]
                a1 = rows_v[r0, 16:32]
                for j in range(1, K_NB):
                    a0 = a0 + rows_v[r0 + j, 0:16]
                    a1 = a1 + rows_v[r0 + j, 16:32]
                out_v[ptk, 0:16] = a0
                out_v[ptk, 16:32] = a1
            pltpu.sync_copy(out_v, out_hbm.at[pl.ds(base, CH)])
            return carry

        lax.fori_loop(0, NCHUNK, chunk, 0)

    return k(idxflat, table)


def _k4_body(h_ref, o_ref):
    h = h_ref[...]
    R, L = h.shape
    lane = lax.broadcasted_iota(jnp.int32, (R, L), 1)
    valid = lane < 18
    even = (lane % 2) == 0
    hv = jnp.where(valid, h, 0.0)
    sq = hv * hv
    s0 = jnp.sum(jnp.where(even, sq, 0.0), axis=1, keepdims=True)
    s1 = jnp.sum(jnp.where(even, 0.0, sq), axis=1, keepdims=True)
    norm = jnp.sqrt(jnp.where(even, s0, s1))
    o_ref[...] = h / jnp.maximum(norm, F32(1e-12))


def _k4(hist):
    BN = hist.shape[0]
    R = 2048
    return pl.pallas_call(
        _k4_body,
        grid=(BN // R,),
        in_specs=[pl.BlockSpec((R, 32), lambda i: (i, 0))],
        out_specs=pl.BlockSpec((R, 32), lambda i: (i, 0)),
        out_shape=jax.ShapeDtypeStruct((BN, 32), F32),
    )(hist)


def _k5_body(h0_ref, w11, b11, w12, b12, w1s, b1s, w21, b21, w22, b22, w2s, b2s, o_ref):
    bf = jnp.bfloat16

    def conv(w_ref, x, b_ref):
        z = lax.dot_general(w_ref[...].astype(bf), x.astype(bf),
                            (((1,), (0,)), ((), ())),
                            preferred_element_type=F32)
        return z + b_ref[...]

    h0 = h0_ref[0]
    t = jnp.maximum(conv(w11, h0, b11), 0.0)
    t = conv(w12, t, b12)
    scp = conv(w1s, h0, b1s)
    h1 = jnp.maximum(t + scp, 0.0)
    t = jnp.maximum(conv(w21, h1, b21), 0.0)
    t = conv(w22, t, b22)
    scp = conv(w2s, h1, b2s)
    o_ref[0] = jnp.maximum(t + scp, 0.0)


def _k5(h0, ws):
    B, _, N = h0.shape
    specs = [pl.BlockSpec((1, 18, N), lambda b: (b, 0, 0))]
    for w in ws:
        specs.append(pl.BlockSpec(w.shape, lambda b: tuple(0 for _ in w.shape)))
    return pl.pallas_call(
        _k5_body,
        grid=(B,),
        in_specs=specs,
        out_specs=pl.BlockSpec((1, 512, N), lambda b: (b, 0, 0)),
        out_shape=jax.ShapeDtypeStruct((B, 512, N), F32),
    )(h0, *ws)


def kernel(x, l1_w1, l1_b1, l1_w2, l1_b2, l1_ws, l1_bs,
           l2_w1, l2_b1, l2_w2, l2_b2, l2_ws, l2_bs):
    B, _, N = x.shape
    BN = B * N
    xt = x.reshape(B, N, 3)
    tx, ty, tz = xt[..., 0], xt[..., 1], xt[..., 2]
    zeros = jnp.zeros_like(tx)
    ptab = jnp.stack([tx, ty, tz, tx * tx, ty * ty, tz * tz,
                      tx * ty, tx * tz, ty * tz,
                      zeros, zeros, zeros, zeros, zeros, zeros, zeros], axis=-1)

    idx, mom = _k1(x, ptab)
    momt = mom.transpose(0, 2, 1).reshape(B, 16, N // 128, 128)
    c18p = _k2(momt)
    c18 = c18p.reshape(B, 32, N).transpose(0, 2, 1).reshape(BN, 32)
    idxflat = idx[:, :, :K_NB].reshape(-1)
    c18pad = jnp.concatenate([c18, jnp.zeros((BN, 96), F32)], axis=1)
    hist = _hist_sc(idxflat, c18pad)
    histn = _k4(hist)
    h0 = histn[:, :18].reshape(B, N * 18).reshape(B, 18, N)
    ws = (l1_w1, l1_b1.reshape(-1, 1), l1_w2, l1_b2.reshape(-1, 1),
          l1_ws, l1_bs.reshape(-1, 1), l2_w1, l2_b1.reshape(-1, 1),
          l2_w2, l2_b2.reshape(-1, 1), l2_ws, l2_bs.reshape(-1, 1))
    return _k5(h0, ws)
